# Initial kernel scaffold; baseline (speedup 1.0000x reference)
import jax
import jax.numpy as jnp
from jax.experimental import pallas as pl
from jax.experimental.pallas import tpu as pltpu
from jax.experimental.pallas import tpu_sc as plsc

_K = 1024
_D = 64
_BETA = 0.25
_BLK = 2048


def _tc_body(z_ref, emb_ref, idx_ref, embn_ref, cb_ref, vq_ref, perp_ref,
             ewn16_scr, e2_scr, counts_scr, loss_scr):
    i = pl.program_id(0)
    nsteps = pl.num_programs(0)

    @pl.when(i == 0)
    def _init():
        ew = emb_ref[...]
        n = jnp.sqrt(jnp.sum(ew * ew, axis=1, keepdims=True))
        ewn = ew / jnp.maximum(n, 1e-12)
        embn_ref[...] = ewn
        ewn16_scr[...] = ewn.astype(jnp.bfloat16)
        e2 = jnp.sum(ewn * ewn, axis=1, keepdims=True)
        e2_scr[...] = e2.T
        counts_scr[...] = jnp.zeros((1, _K), jnp.float32)
        loss_scr[0] = 0.0

    z = z_ref[...]
    zn = z / jnp.maximum(jnp.sqrt(jnp.sum(z * z, axis=1, keepdims=True)), 1e-12)
    dot = jax.lax.dot_general(
        zn.astype(jnp.bfloat16), ewn16_scr[...],
        (((1,), (1,)), ((), ())), preferred_element_type=jnp.float32)
    score = e2_scr[...] - 2.0 * dot
    m = jnp.min(score, axis=1, keepdims=True)
    iota = jax.lax.broadcasted_iota(jnp.int32, score.shape, 1)
    idx = jnp.min(jnp.where(score == m, iota, _K), axis=1, keepdims=True)
    idx_ref[...] = idx

    onehot = (iota == idx).astype(jnp.float32)
    counts_scr[...] += jnp.sum(onehot, axis=0, keepdims=True)
    znorm2 = jnp.sum(zn * zn, axis=1, keepdims=True)
    loss_scr[0] += jnp.sum(znorm2 + m)

    @pl.when(i == nsteps - 1)
    def _fini():
        total_rows = nsteps * _BLK
        cb = loss_scr[0] / (total_rows * _D)
        p = counts_scr[...] * (1.0 / total_rows)
        ent = -jnp.sum(p * jnp.log(p + 1e-10))
        ones = jnp.ones((1, 1), jnp.float32)
        cb_ref[...] = cb * ones
        vq_ref[...] = (cb + _BETA * cb) * ones
        perp_ref[...] = jnp.exp(ent) * ones


def _tc_stage(z_flat, emb_weight, interpret=False):
    n_rows = z_flat.shape[0]
    grid = n_rows // _BLK
    return pl.pallas_call(
        _tc_body,
        grid=(grid,),
        in_specs=[
            pl.BlockSpec((_BLK, _D), lambda i: (i, 0)),
            pl.BlockSpec((_K, _D), lambda i: (0, 0)),
        ],
        out_specs=[
            pl.BlockSpec((_BLK, 1), lambda i: (i, 0)),
            pl.BlockSpec((_K, _D), lambda i: (0, 0)),
            pl.BlockSpec((1, 1), lambda i: (0, 0)),
            pl.BlockSpec((1, 1), lambda i: (0, 0)),
            pl.BlockSpec((1, 1), lambda i: (0, 0)),
        ],
        out_shape=[
            jax.ShapeDtypeStruct((n_rows, 1), jnp.int32),
            jax.ShapeDtypeStruct((_K, _D), jnp.float32),
            jax.ShapeDtypeStruct((1, 1), jnp.float32),
            jax.ShapeDtypeStruct((1, 1), jnp.float32),
            jax.ShapeDtypeStruct((1, 1), jnp.float32),
        ],
        scratch_shapes=[
            pltpu.VMEM((_K, _D), jnp.bfloat16),
            pltpu.VMEM((1, _K), jnp.float32),
            pltpu.VMEM((1, _K), jnp.float32),
            pltpu.SMEM((1,), jnp.float32),
        ],
        interpret=interpret,
    )(z_flat, emb_weight)


_GATHER_WIN = 128


def _sc_gather(table, indices_2d, n_rows):
    mesh = plsc.VectorSubcoreMesh(core_axis_name="core",
                                  subcore_axis_name="subcore")

    @pl.kernel(out_type=jax.ShapeDtypeStruct((n_rows, _D), table.dtype),
               mesh=mesh)
    def k(x_hbm, i_hbm, o_hbm):
        def body(i_vmem, o_vmem):
            pltpu.sync_copy(x_hbm.at[i_vmem.at[0]], o_vmem)

        pltpu.emit_pipeline(
            body,
            grid=(n_rows // _GATHER_WIN,),
            in_specs=[pl.BlockSpec((1, _GATHER_WIN), index_map=lambda i: (0, i))],
            out_specs=[pl.BlockSpec((_GATHER_WIN, _D), index_map=lambda i: (i, 0))],
            core_axis_name=("core", "subcore"),
            dimension_semantics=(pltpu.PARALLEL,),
        )(i_hbm, o_hbm)

    return k(table, indices_2d)


def kernel(z_e, emb_weight):
    n_rows = z_e.shape[0] * z_e.shape[1]
    z_flat = z_e.reshape(n_rows, _D)
    idx2d, embn, cb, vq, perp = _tc_stage(z_flat, emb_weight)
    z_q = _sc_gather(embn, idx2d.reshape(1, n_rows), n_rows)
    z_q = z_q.reshape(z_e.shape)
    cb_s = cb[0, 0]
    return (z_q, vq[0, 0], cb_s, cb_s, perp[0, 0], idx2d.reshape(n_rows))


# speedup vs baseline: 1.7808x; 1.7808x over previous
import jax
import jax.numpy as jnp
from jax.experimental import pallas as pl
from jax.experimental.pallas import tpu as pltpu
from jax.experimental.pallas import tpu_sc as plsc

_K = 1024
_D = 64
_BETA = 0.25
_BLK = 2048


def _tc_body(z_ref, emb_ref, idx_ref, embn_ref, cb_ref, vq_ref, perp_ref,
             ewn16_scr, e2_scr, counts_scr, loss_scr):
    i = pl.program_id(0)
    nsteps = pl.num_programs(0)

    @pl.when(i == 0)
    def _init():
        ew = emb_ref[...]
        n = jnp.sqrt(jnp.sum(ew * ew, axis=1, keepdims=True))
        ewn = ew / jnp.maximum(n, 1e-12)
        embn_ref[...] = jnp.concatenate(
            [ewn, jnp.zeros((_K, 128 - _D), jnp.float32)], axis=1)
        ewn16_scr[...] = ewn.astype(jnp.bfloat16)
        e2 = jnp.sum(ewn * ewn, axis=1, keepdims=True)
        e2_scr[...] = e2.T
        counts_scr[...] = jnp.zeros((1, _K), jnp.float32)
        loss_scr[0] = 0.0

    z = z_ref[...]
    zn = z / jnp.maximum(jnp.sqrt(jnp.sum(z * z, axis=1, keepdims=True)), 1e-12)
    dot = jax.lax.dot_general(
        zn.astype(jnp.bfloat16), ewn16_scr[...],
        (((1,), (1,)), ((), ())), preferred_element_type=jnp.float32)
    score = e2_scr[...] - 2.0 * dot
    m = jnp.min(score, axis=1, keepdims=True)
    iota = jax.lax.broadcasted_iota(jnp.int32, score.shape, 1)
    idx = jnp.min(jnp.where(score == m, iota, _K), axis=1, keepdims=True)
    idx_ref[...] = idx

    onehot = (iota == idx).astype(jnp.float32)
    counts_scr[...] += jnp.sum(onehot, axis=0, keepdims=True)
    znorm2 = jnp.sum(zn * zn, axis=1, keepdims=True)
    loss_scr[0] += jnp.sum(znorm2 + m)

    @pl.when(i == nsteps - 1)
    def _fini():
        total_rows = nsteps * _BLK
        cb = loss_scr[0] / (total_rows * _D)
        p = counts_scr[...] * (1.0 / total_rows)
        ent = -jnp.sum(p * jnp.log(p + 1e-10))
        ones = jnp.ones((1, 1), jnp.float32)
        cb_ref[...] = cb * ones
        vq_ref[...] = (cb + _BETA * cb) * ones
        perp_ref[...] = jnp.exp(ent) * ones


def _tc_stage(z_flat, emb_weight, interpret=False):
    n_rows = z_flat.shape[0]
    grid = n_rows // _BLK
    return pl.pallas_call(
        _tc_body,
        grid=(grid,),
        in_specs=[
            pl.BlockSpec((_BLK, _D), lambda i: (i, 0)),
            pl.BlockSpec((_K, _D), lambda i: (0, 0)),
        ],
        out_specs=[
            pl.BlockSpec((_BLK, 1), lambda i: (i, 0)),
            pl.BlockSpec((_K, 128), lambda i: (0, 0)),
            pl.BlockSpec((1, 1), lambda i: (0, 0)),
            pl.BlockSpec((1, 1), lambda i: (0, 0)),
            pl.BlockSpec((1, 1), lambda i: (0, 0)),
        ],
        out_shape=[
            jax.ShapeDtypeStruct((n_rows, 1), jnp.int32),
            jax.ShapeDtypeStruct((_K, 128), jnp.float32),
            jax.ShapeDtypeStruct((1, 1), jnp.float32),
            jax.ShapeDtypeStruct((1, 1), jnp.float32),
            jax.ShapeDtypeStruct((1, 1), jnp.float32),
        ],
        scratch_shapes=[
            pltpu.VMEM((_K, _D), jnp.bfloat16),
            pltpu.VMEM((1, _K), jnp.float32),
            pltpu.VMEM((1, _K), jnp.float32),
            pltpu.SMEM((1,), jnp.float32),
        ],
        interpret=interpret,
    )(z_flat, emb_weight)


_GATHER_WIN = 128


def _sc_gather(table, indices_2d, n_rows):
    mesh = plsc.VectorSubcoreMesh(core_axis_name="core",
                                  subcore_axis_name="subcore")

    @pl.kernel(out_type=jax.ShapeDtypeStruct((n_rows, 128), table.dtype),
               mesh=mesh)
    def k(x_hbm, i_hbm, o_hbm):
        def body(i_vmem, o_vmem):
            pltpu.sync_copy(x_hbm.at[i_vmem.at[0]], o_vmem)

        pltpu.emit_pipeline(
            body,
            grid=(n_rows // _GATHER_WIN,),
            in_specs=[pl.BlockSpec((1, _GATHER_WIN), index_map=lambda i: (0, i))],
            out_specs=[pl.BlockSpec((_GATHER_WIN, 128), index_map=lambda i: (i, 0))],
            core_axis_name=("core", "subcore"),
            dimension_semantics=(pltpu.PARALLEL,),
        )(i_hbm, o_hbm)

    return k(table, indices_2d)


def kernel(z_e, emb_weight):
    n_rows = z_e.shape[0] * z_e.shape[1]
    z_flat = z_e.reshape(n_rows, _D)
    idx2d, embn, cb, vq, perp = _tc_stage(z_flat, emb_weight)
    z_q = _sc_gather(embn, idx2d.reshape(1, n_rows), n_rows)
    z_q = z_q[:, :_D].reshape(z_e.shape)
    cb_s = cb[0, 0]
    return (z_q, vq[0, 0], cb_s, cb_s, perp[0, 0], idx2d.reshape(n_rows))
